# baseline (speedup 1.0000x reference)
import functools

import jax
import jax.numpy as jnp
from jax import lax
from jax.experimental import pallas as pl
from jax.experimental.pallas import tpu as pltpu
from jax.experimental.pallas import tpu_sc as plsc

NUM_TOKENS_C = 16384
CODE_DIM_C = 256
NUM_EMB_C = 8192

PREP_BLOCK = 1024
TOK_BLOCK = 256
COL_CHUNK = 4096
SC_CHUNK = 128


def _argmin_body(ah_ref, sxn_ref, wht2_ref, cn2_ref, idx_ref):
    ah = ah_ref[...]
    sxn = sxn_ref[...][:, None]

    run_val = jnp.full((TOK_BLOCK, 1), jnp.inf, dtype=jnp.float32)
    run_idx = jnp.zeros((TOK_BLOCK, 1), dtype=jnp.int32)
    for j in range(NUM_EMB_C // COL_CHUNK):
        wht2 = wht2_ref[:, pl.ds(j * COL_CHUNK, COL_CHUNK)]
        cn2 = cn2_ref[pl.ds(j * COL_CHUNK, COL_CHUNK)]
        m2 = lax.dot_general(
            ah, wht2, (((1,), (0,)), ((), ())),
            preferred_element_type=jnp.float32,
        )
        d = (sxn + cn2[None, :]) - m2
        dmin = jnp.min(d, axis=1, keepdims=True)
        ii = lax.broadcasted_iota(jnp.int32, (TOK_BLOCK, COL_CHUNK), 1)
        ic = jnp.min(
            jnp.where(d == dmin, ii, jnp.int32(2**30)),
            axis=1, keepdims=True,
        ) + jnp.int32(j * COL_CHUNK)
        upd = dmin < run_val
        run_idx = jnp.where(upd, ic, run_idx)
        run_val = jnp.where(upd, dmin, run_val)
        run_val = run_val.astype(jnp.bfloat16).astype(jnp.float32)
    idx_ref[...] = run_idx[:, 0]


def _sc_gather(w, idx, x):
    info = plsc.get_sparse_core_info()
    nw = info.num_cores * info.num_subcores
    tok_per_w = NUM_TOKENS_C // nw
    n_chunks = tok_per_w // SC_CHUNK
    mesh = plsc.VectorSubcoreMesh(core_axis_name="c", subcore_axis_name="s")

    @functools.partial(
        pl.kernel,
        mesh=mesh,
        out_type=[
            jax.ShapeDtypeStruct((NUM_TOKENS_C, CODE_DIM_C), jnp.float32),
            jax.ShapeDtypeStruct((nw, 16), jnp.float32),
        ],
        scratch_types=[
            pltpu.VMEM((SC_CHUNK,), jnp.int32),
            pltpu.VMEM((SC_CHUNK, CODE_DIM_C), jnp.float32),
            pltpu.VMEM((SC_CHUNK, CODE_DIM_C), jnp.float32),
            pltpu.VMEM((16,), jnp.float32),
            pltpu.SemaphoreType.DMA,
        ],
    )
    def body(w_hbm, idx_hbm, x_hbm, ste_hbm, loss_hbm,
             idx_v, rows_v, x_v, acc_v, sem):
        wid = lax.axis_index("s") * info.num_cores + lax.axis_index("c")
        base = wid * tok_per_w

        def chunk_body(ch, acc):
            off = base + ch * SC_CHUNK
            pltpu.sync_copy(idx_hbm.at[pl.ds(off, SC_CHUNK)], idx_v)
            gcp = pltpu.async_copy(w_hbm.at[idx_v], rows_v, sem)
            pltpu.sync_copy(x_hbm.at[pl.ds(off, SC_CHUNK), :], x_v)
            gcp.wait()

            def row_body(i, a):
                for k in range(CODE_DIM_C // 16):
                    sl = pl.ds(k * 16, 16)
                    q = rows_v[i, sl]
                    xv = x_v[i, sl]
                    dq = q - xv
                    a = a + dq * dq
                    rows_v[i, sl] = xv + dq
                return a

            acc = lax.fori_loop(0, SC_CHUNK, row_body, acc)
            pltpu.sync_copy(rows_v, ste_hbm.at[pl.ds(off, SC_CHUNK), :])
            return acc

        acc = lax.fori_loop(
            0, n_chunks, chunk_body, jnp.zeros((16,), jnp.float32)
        )
        acc_v[...] = acc
        pltpu.sync_copy(acc_v, loss_hbm.at[wid])

    return body(w, idx, x)


def _normalize(v):
    norm = jnp.linalg.norm(v, ord=2, axis=-1, keepdims=True)
    return v / jnp.maximum(norm, 1e-12)


def kernel(x, W):
    xn = _normalize(x)
    wn = _normalize(W)
    sxn = jnp.sum(xn ** 2, axis=1)
    cn2 = jnp.sum(wn ** 2, axis=1)

    ah = xn.astype(jnp.bfloat16)
    wht2 = (wn.astype(jnp.bfloat16) * jnp.bfloat16(2.0)).T

    idx = pl.pallas_call(
        _argmin_body,
        grid=(NUM_TOKENS_C // TOK_BLOCK,),
        in_specs=[
            pl.BlockSpec((TOK_BLOCK, CODE_DIM_C), lambda i: (i, 0)),
            pl.BlockSpec((TOK_BLOCK,), lambda i: (i,)),
            pl.BlockSpec((CODE_DIM_C, NUM_EMB_C), lambda i: (0, 0)),
            pl.BlockSpec((NUM_EMB_C,), lambda i: (0,)),
        ],
        out_specs=pl.BlockSpec((TOK_BLOCK,), lambda i: (i,)),
        out_shape=jax.ShapeDtypeStruct((NUM_TOKENS_C,), jnp.int32),
    )(ah, sxn, wht2, cn2)

    ste, loss_part = _sc_gather(W, idx, x)
    loss = jnp.sum(loss_part) / jnp.float32(NUM_TOKENS_C * CODE_DIM_C)
    return ste, loss, loss, idx


# speedup vs baseline: 1.0074x; 1.0074x over previous
import functools

import jax
import jax.numpy as jnp
from jax import lax
from jax.experimental import pallas as pl
from jax.experimental.pallas import tpu as pltpu
from jax.experimental.pallas import tpu_sc as plsc

NUM_TOKENS_C = 16384
CODE_DIM_C = 256
NUM_EMB_C = 8192

PREP_BLOCK = 1024
TOK_BLOCK = 512
COL_CHUNK = 4096
SC_CHUNK = 128


def _argmin_body(ah_ref, sxn_ref, wht2_ref, cn2_ref, idx_ref):
    ah = ah_ref[...]
    sxn = sxn_ref[...][:, None]

    run_val = jnp.full((TOK_BLOCK, 1), jnp.inf, dtype=jnp.float32)
    run_idx = jnp.zeros((TOK_BLOCK, 1), dtype=jnp.int32)
    for j in range(NUM_EMB_C // COL_CHUNK):
        wht2 = wht2_ref[:, pl.ds(j * COL_CHUNK, COL_CHUNK)]
        cn2 = cn2_ref[pl.ds(j * COL_CHUNK, COL_CHUNK)]
        m2 = lax.dot_general(
            ah, wht2, (((1,), (0,)), ((), ())),
            preferred_element_type=jnp.float32,
        )
        d = (sxn + cn2[None, :]) - m2
        dmin = jnp.min(d, axis=1, keepdims=True)
        ii = lax.broadcasted_iota(jnp.int32, (TOK_BLOCK, COL_CHUNK), 1)
        ic = jnp.min(
            jnp.where(d == dmin, ii, jnp.int32(2**30)),
            axis=1, keepdims=True,
        ) + jnp.int32(j * COL_CHUNK)
        upd = dmin < run_val
        run_idx = jnp.where(upd, ic, run_idx)
        run_val = jnp.where(upd, dmin, run_val)
        run_val = run_val.astype(jnp.bfloat16).astype(jnp.float32)
    idx_ref[...] = run_idx[:, 0]


def _sc_gather(w, idx, x):
    info = plsc.get_sparse_core_info()
    nw = info.num_cores * info.num_subcores
    tok_per_w = NUM_TOKENS_C // nw
    n_chunks = tok_per_w // SC_CHUNK
    mesh = plsc.VectorSubcoreMesh(core_axis_name="c", subcore_axis_name="s")

    @functools.partial(
        pl.kernel,
        mesh=mesh,
        out_type=[
            jax.ShapeDtypeStruct((NUM_TOKENS_C, CODE_DIM_C), jnp.float32),
            jax.ShapeDtypeStruct((nw, 16), jnp.float32),
        ],
        scratch_types=[
            pltpu.VMEM((SC_CHUNK,), jnp.int32),
            pltpu.VMEM((SC_CHUNK, CODE_DIM_C), jnp.float32),
            pltpu.VMEM((SC_CHUNK, CODE_DIM_C), jnp.float32),
            pltpu.VMEM((16,), jnp.float32),
            pltpu.SemaphoreType.DMA,
        ],
    )
    def body(w_hbm, idx_hbm, x_hbm, ste_hbm, loss_hbm,
             idx_v, rows_v, x_v, acc_v, sem):
        wid = lax.axis_index("s") * info.num_cores + lax.axis_index("c")
        base = wid * tok_per_w

        def chunk_body(ch, acc):
            off = base + ch * SC_CHUNK
            pltpu.sync_copy(idx_hbm.at[pl.ds(off, SC_CHUNK)], idx_v)
            gcp = pltpu.async_copy(w_hbm.at[idx_v], rows_v, sem)
            pltpu.sync_copy(x_hbm.at[pl.ds(off, SC_CHUNK), :], x_v)
            gcp.wait()

            def row_body(i, a):
                for k in range(CODE_DIM_C // 16):
                    sl = pl.ds(k * 16, 16)
                    q = rows_v[i, sl]
                    xv = x_v[i, sl]
                    dq = q - xv
                    a = a + dq * dq
                    rows_v[i, sl] = xv + dq
                return a

            acc = lax.fori_loop(0, SC_CHUNK, row_body, acc)
            pltpu.sync_copy(rows_v, ste_hbm.at[pl.ds(off, SC_CHUNK), :])
            return acc

        acc = lax.fori_loop(
            0, n_chunks, chunk_body, jnp.zeros((16,), jnp.float32)
        )
        acc_v[...] = acc
        pltpu.sync_copy(acc_v, loss_hbm.at[wid])

    return body(w, idx, x)


def _normalize(v):
    norm = jnp.linalg.norm(v, ord=2, axis=-1, keepdims=True)
    return v / jnp.maximum(norm, 1e-12)


def kernel(x, W):
    xn = _normalize(x)
    wn = _normalize(W)
    sxn = jnp.sum(xn ** 2, axis=1)
    cn2 = jnp.sum(wn ** 2, axis=1)

    ah = xn.astype(jnp.bfloat16)
    wht2 = (wn.astype(jnp.bfloat16) * jnp.bfloat16(2.0)).T

    idx = pl.pallas_call(
        _argmin_body,
        grid=(NUM_TOKENS_C // TOK_BLOCK,),
        in_specs=[
            pl.BlockSpec((TOK_BLOCK, CODE_DIM_C), lambda i: (i, 0)),
            pl.BlockSpec((TOK_BLOCK,), lambda i: (i,)),
            pl.BlockSpec((CODE_DIM_C, NUM_EMB_C), lambda i: (0, 0)),
            pl.BlockSpec((NUM_EMB_C,), lambda i: (0,)),
        ],
        out_specs=pl.BlockSpec((TOK_BLOCK,), lambda i: (i,)),
        out_shape=jax.ShapeDtypeStruct((NUM_TOKENS_C,), jnp.int32),
    )(ah, sxn, wht2, cn2)

    ste, loss_part = _sc_gather(W, idx, x)
    loss = jnp.sum(loss_part) / jnp.float32(NUM_TOKENS_C * CODE_DIM_C)
    return ste, loss, loss, idx


# speedup vs baseline: 1.0307x; 1.0231x over previous
import functools

import jax
import jax.numpy as jnp
from jax import lax
from jax.experimental import pallas as pl
from jax.experimental.pallas import tpu as pltpu
from jax.experimental.pallas import tpu_sc as plsc

NUM_TOKENS_C = 16384
CODE_DIM_C = 256
NUM_EMB_C = 8192

PREP_BLOCK = 1024
TOK_BLOCK = 256
COL_CHUNK = 4096
SC_CHUNK = 128


def _argmin_body(ah_ref, sxn_ref, wht_ref, cn2_ref, idx_ref):
    ah = ah_ref[...]
    sxn = sxn_ref[...][:, None]

    run_val = jnp.full((TOK_BLOCK, 1), jnp.inf, dtype=jnp.float32)
    run_idx = jnp.zeros((TOK_BLOCK, 1), dtype=jnp.int32)
    for j in range(NUM_EMB_C // COL_CHUNK):
        wht = wht_ref[:, pl.ds(j * COL_CHUNK, COL_CHUNK)]
        cn2 = cn2_ref[pl.ds(j * COL_CHUNK, COL_CHUNK)]
        m = lax.dot_general(
            ah, wht, (((1,), (0,)), ((), ())),
            preferred_element_type=jnp.float32,
        )
        d = (sxn + cn2[None, :]) - 2.0 * m
        dmin = jnp.min(d, axis=1, keepdims=True)
        ii = lax.broadcasted_iota(jnp.int32, (TOK_BLOCK, COL_CHUNK), 1)
        ic = jnp.min(
            jnp.where(d == dmin, ii + j * COL_CHUNK, jnp.int32(2**30)),
            axis=1, keepdims=True,
        )
        upd = dmin < run_val
        run_idx = jnp.where(upd, ic, run_idx)
        run_val = jnp.where(upd, dmin, run_val)
        run_val = run_val.astype(jnp.bfloat16).astype(jnp.float32)
    idx_ref[...] = run_idx[:, 0]


def _sc_gather(w, idx, x):
    info = plsc.get_sparse_core_info()
    nw = info.num_cores * info.num_subcores
    tok_per_w = NUM_TOKENS_C // nw
    n_chunks = tok_per_w // SC_CHUNK
    mesh = plsc.VectorSubcoreMesh(core_axis_name="c", subcore_axis_name="s")

    @functools.partial(
        pl.kernel,
        mesh=mesh,
        out_type=[
            jax.ShapeDtypeStruct((NUM_TOKENS_C, CODE_DIM_C), jnp.float32),
            jax.ShapeDtypeStruct((nw, 16), jnp.float32),
        ],
        scratch_types=[
            pltpu.VMEM((SC_CHUNK,), jnp.int32),
            pltpu.VMEM((SC_CHUNK, CODE_DIM_C), jnp.float32),
            pltpu.VMEM((SC_CHUNK, CODE_DIM_C), jnp.float32),
            pltpu.VMEM((16,), jnp.float32),
            pltpu.SemaphoreType.DMA,
        ],
    )
    def body(w_hbm, idx_hbm, x_hbm, ste_hbm, loss_hbm,
             idx_v, rows_v, x_v, acc_v, sem):
        wid = lax.axis_index("s") * info.num_cores + lax.axis_index("c")
        base = wid * tok_per_w

        def chunk_body(ch, acc):
            off = base + ch * SC_CHUNK
            pltpu.sync_copy(idx_hbm.at[pl.ds(off, SC_CHUNK)], idx_v)
            gcp = pltpu.async_copy(w_hbm.at[idx_v], rows_v, sem)
            pltpu.sync_copy(x_hbm.at[pl.ds(off, SC_CHUNK), :], x_v)
            gcp.wait()

            def row_body(i, a):
                for k in range(CODE_DIM_C // 16):
                    sl = pl.ds(k * 16, 16)
                    q = rows_v[i, sl]
                    xv = x_v[i, sl]
                    dq = q - xv
                    a = a + dq * dq
                    rows_v[i, sl] = xv + dq
                return a

            acc = lax.fori_loop(0, SC_CHUNK, row_body, acc)
            pltpu.sync_copy(rows_v, ste_hbm.at[pl.ds(off, SC_CHUNK), :])
            return acc

        acc = lax.fori_loop(
            0, n_chunks, chunk_body, jnp.zeros((16,), jnp.float32)
        )
        acc_v[...] = acc
        pltpu.sync_copy(acc_v, loss_hbm.at[wid])

    return body(w, idx, x)


def _normalize(v):
    norm = jnp.linalg.norm(v, ord=2, axis=-1, keepdims=True)
    return v / jnp.maximum(norm, 1e-12)


def kernel(x, W):
    xn = _normalize(x)
    wn = _normalize(W)
    sxn = jnp.sum(xn ** 2, axis=1)
    cn2 = jnp.sum(wn ** 2, axis=1)

    ah = xn.astype(jnp.bfloat16)
    wht = wn.astype(jnp.bfloat16).T

    idx = pl.pallas_call(
        _argmin_body,
        grid=(NUM_TOKENS_C // TOK_BLOCK,),
        in_specs=[
            pl.BlockSpec((TOK_BLOCK, CODE_DIM_C), lambda i: (i, 0)),
            pl.BlockSpec((TOK_BLOCK,), lambda i: (i,)),
            pl.BlockSpec((CODE_DIM_C, NUM_EMB_C), lambda i: (0, 0)),
            pl.BlockSpec((NUM_EMB_C,), lambda i: (0,)),
        ],
        out_specs=pl.BlockSpec((TOK_BLOCK,), lambda i: (i,)),
        out_shape=jax.ShapeDtypeStruct((NUM_TOKENS_C,), jnp.int32),
    )(ah, sxn, wht, cn2)

    ste, loss_part = _sc_gather(W, idx, x)
    loss = jnp.sum(loss_part) / jnp.float32(NUM_TOKENS_C * CODE_DIM_C)
    return ste, loss, loss, idx
